# baseline (speedup 1.0000x reference)
import functools

import jax
import jax.numpy as jnp
from jax import lax
from jax.experimental import pallas as pl
from jax.experimental.pallas import tpu as pltpu
from jax.experimental.pallas import tpu_sc as plsc

N_NODES = 10000
N_PAD = 10240
N_EDGES = 320000
NCORES = 2
NSUB = 16
CHUNK = 96
RPS = N_PAD // NSUB

_MESH = dict(core_axis_name="c", subcore_axis_name="s",
             num_cores=NCORES, num_subcores=NSUB)

_E_TILE = N_EDGES // (NCORES * NSUB)
_ET_CHUNKS = -(-_E_TILE // CHUNK)
_ET_REAL_LAST = _E_TILE - (_ET_CHUNKS - 1) * CHUNK
_E_SUB = N_EDGES // NSUB
_ES_CHUNKS = -(-_E_SUB // CHUNK)
_ES_REAL_LAST = _E_SUB - (_ES_CHUNKS - 1) * CHUNK
_NB = 3


def _fill_dummy(idx_ref, nreal, value):
    vec = jnp.full((16,), value, jnp.int32)
    for j in range(nreal // 16, CHUNK // 16):
        idx_ref[pl.ds(j * 16, 16)] = vec


@functools.partial(
    pl.kernel,
    out_type=(jax.ShapeDtypeStruct((2 * N_PAD, 128), jnp.float32),
              jax.ShapeDtypeStruct((2 * N_PAD,), jnp.float32)),
    mesh=plsc.VectorSubcoreMesh(**_MESH),
    scratch_types=[
        [pltpu.VMEM((CHUNK,), jnp.int32)] * _NB,
        [pltpu.VMEM((CHUNK,), jnp.int32)] * _NB,
        [pltpu.VMEM((CHUNK, 128), jnp.float32)] * _NB,
        pltpu.VMEM((CHUNK,), jnp.float32),
        pltpu.VMEM((RPS,), jnp.float32),
        pltpu.VMEM_SHARED((N_PAD, 128), jnp.float32),
        pltpu.VMEM_SHARED((N_PAD,), jnp.float32),
        [pltpu.SemaphoreType.DMA] * _NB,
        [pltpu.SemaphoreType.DMA] * _NB,
        [pltpu.SemaphoreType.DMA] * _NB,
    ],
)
def _agg_edge(tab, src_h, dst_h, z_h, out, dout,
              src_v, dst_v, rows_v, ones_v, z1_v, acc, dacc,
              gsem, ssem, dsem):
    core = lax.axis_index("c")
    sub = lax.axis_index("s")
    wid = core * NSUB + sub
    zvec = jnp.zeros((16,), jnp.float32)
    ovec = jnp.ones((16,), jnp.float32)
    for j in range(CHUNK // 16):
        ones_v[pl.ds(j * 16, 16)] = ovec
    for j in range(RPS // 16):
        z1_v[pl.ds(j * 16, 16)] = zvec
    pltpu.sync_copy(z_h, acc.at[pl.ds(sub * RPS, RPS)])
    pltpu.sync_copy(z1_v, dacc.at[pl.ds(sub * RPS, RPS)])
    plsc.subcore_barrier()

    def load_and_gather(c, b, nreal):
        base = wid * _E_TILE + c * CHUNK
        if nreal == CHUNK:
            pltpu.sync_copy(src_h.at[pl.ds(base, CHUNK)], src_v[b])
            pltpu.sync_copy(dst_h.at[pl.ds(base, CHUNK)], dst_v[b])
        else:
            pltpu.sync_copy(src_h.at[pl.ds(base, nreal)],
                            src_v[b].at[pl.ds(0, nreal)])
            pltpu.sync_copy(dst_h.at[pl.ds(base, nreal)],
                            dst_v[b].at[pl.ds(0, nreal)])
            _fill_dummy(src_v[b], nreal, 0)
            _fill_dummy(dst_v[b], nreal, N_NODES)
        pltpu.async_copy(tab.at[src_v[b]], rows_v[b], gsem[b])

    def consume(c, b):
        pltpu.make_async_copy(tab.at[src_v[b]], rows_v[b], gsem[b]).wait()
        pltpu.async_copy(rows_v[b], acc.at[dst_v[b]], ssem[b], add=True)
        pltpu.async_copy(ones_v, dacc.at[dst_v[b]], dsem[b], add=True)

    def drain(b):
        pltpu.make_async_copy(rows_v[b], acc.at[dst_v[b]], ssem[b]).wait()
        pltpu.make_async_copy(ones_v, dacc.at[dst_v[b]], dsem[b]).wait()

    load_and_gather(0, 0, CHUNK)
    load_and_gather(1, 1, CHUNK)

    def step(p, carry):
        for b in range(_NB):
            c = p * _NB + b
            bn = (b + 2) % _NB
            consume(c, b)
            if b == 0:
                @pl.when(p == _ET_CHUNKS // _NB - 1)
                def _refill_last():
                    @pl.when(c >= 1)
                    def _d():
                        drain(bn)
                    load_and_gather(c + 2, bn, _ET_REAL_LAST)

                @pl.when(p < _ET_CHUNKS // _NB - 1)
                def _refill0():
                    @pl.when(c >= 1)
                    def _d():
                        drain(bn)
                    load_and_gather(c + 2, bn, CHUNK)
            else:
                @pl.when(c + 2 <= _ET_CHUNKS - 1)
                def _refill():
                    drain(bn)
                    load_and_gather(c + 2, bn, CHUNK)

        return carry

    lax.fori_loop(0, _ET_CHUNKS // _NB, step, 0)
    for b in range(_NB):
        drain(b)
    plsc.subcore_barrier()
    pltpu.sync_copy(acc.at[pl.ds(sub * RPS, RPS)],
                    out.at[pl.ds(core * N_PAD + sub * RPS, RPS)])
    pltpu.sync_copy(dacc.at[pl.ds(sub * RPS, RPS)],
                    dout.at[pl.ds(core * N_PAD + sub * RPS, RPS)])


@functools.partial(
    pl.kernel,
    out_type=jax.ShapeDtypeStruct((2 * N_PAD, 128), jnp.float32),
    mesh=plsc.VectorSubcoreMesh(**_MESH),
    scratch_types=[
        [pltpu.VMEM((CHUNK,), jnp.int32)] * _NB,
        [pltpu.VMEM((CHUNK,), jnp.int32)] * _NB,
        [pltpu.VMEM((CHUNK, 128), jnp.float32)] * _NB,
        pltpu.VMEM_SHARED((N_PAD, 128), jnp.float32),
        [pltpu.SemaphoreType.DMA] * _NB,
        [pltpu.SemaphoreType.DMA] * _NB,
    ],
)
def _agg_col(tab, src_h, dst_h, z_h, out,
             src_v, dst_v, rows_v, acc, gsem, ssem):
    core = lax.axis_index("c")
    sub = lax.axis_index("s")
    toff = core * N_PAD
    pltpu.sync_copy(z_h, acc.at[pl.ds(sub * RPS, RPS)])
    plsc.subcore_barrier()

    def load_and_gather(c, b, nreal):
        base = sub * _E_SUB + c * CHUNK
        if nreal == CHUNK:
            pltpu.sync_copy(src_h.at[pl.ds(base, CHUNK)], src_v[b])
            pltpu.sync_copy(dst_h.at[pl.ds(base, CHUNK)], dst_v[b])
        else:
            pltpu.sync_copy(src_h.at[pl.ds(base, nreal)],
                            src_v[b].at[pl.ds(0, nreal)])
            pltpu.sync_copy(dst_h.at[pl.ds(base, nreal)],
                            dst_v[b].at[pl.ds(0, nreal)])
            _fill_dummy(src_v[b], nreal, 0)
            _fill_dummy(dst_v[b], nreal, N_NODES)
        for j in range(CHUNK // 16):
            sl = pl.ds(j * 16, 16)
            src_v[b][sl] = src_v[b][sl] + toff
        pltpu.async_copy(tab.at[src_v[b]], rows_v[b], gsem[b])

    def consume(c, b):
        pltpu.make_async_copy(tab.at[src_v[b]], rows_v[b], gsem[b]).wait()
        pltpu.async_copy(rows_v[b], acc.at[dst_v[b]], ssem[b], add=True)

    def drain(b):
        pltpu.make_async_copy(rows_v[b], acc.at[dst_v[b]], ssem[b]).wait()

    load_and_gather(0, 0, CHUNK)
    load_and_gather(1, 1, CHUNK)

    def step(p, carry):
        for b in range(_NB):
            c = p * _NB + b
            bn = (b + 2) % _NB
            consume(c, b)
            if b == 2:
                @pl.when(p == _ES_CHUNKS // _NB - 1)
                def _refill_last():
                    drain(bn)
                    load_and_gather(c + 2, bn, _ES_REAL_LAST)

                @pl.when(p < _ES_CHUNKS // _NB - 1)
                def _refill2():
                    drain(bn)
                    load_and_gather(c + 2, bn, CHUNK)
            else:
                @pl.when(c >= 1)
                def _d():
                    drain(bn)

                @pl.when(c + 2 <= _ES_CHUNKS - 1)
                def _refill():
                    load_and_gather(c + 2, bn, CHUNK)

        return carry

    lax.fori_loop(0, _ES_CHUNKS // _NB, step, 0)
    consume(_ES_CHUNKS - 2, 0)
    drain(2)
    consume(_ES_CHUNKS - 1, 1)
    drain(0)
    drain(1)
    plsc.subcore_barrier()
    pltpu.sync_copy(acc.at[pl.ds(sub * RPS, RPS)],
                    out.at[pl.ds(core * N_PAD + sub * RPS, RPS)])


_BLK = 640
_GRID = N_PAD // _BLK


def _dense(x, nmean, Ws_ref, Wn_ref, sc_ref, sh_ref):
    y = (jnp.dot(x, Ws_ref[...], preferred_element_type=jnp.float32)
         + jnp.dot(nmean, Wn_ref[...], preferred_element_type=jnp.float32))
    y = y * sc_ref[...] + sh_ref[...]
    return jnp.where(y >= 0, y, 0.01 * y)


def _rdeg(dg_ref):
    deg = dg_ref[0, :] + dg_ref[1, :]
    return 1.0 / jnp.maximum(deg, 1.0)


def _layer1_body(x_ref, ss_ref, dg_ref, Ws_ref, Wn_ref,
                 sc_ref, sh_ref, out_ref):
    nmean = (ss_ref[0] + ss_ref[1]) * _rdeg(dg_ref)[:, None]
    act = _dense(x_ref[...], nmean, Ws_ref, Wn_ref, sc_ref, sh_ref)
    out_ref[0] = act[:, :128]
    out_ref[1] = act[:, 128:]


def _layer2_body(x_ref, ss_ref, dg_ref, Ws_ref, Wn_ref,
                 sc_ref, sh_ref, out_ref):
    x = jnp.concatenate([x_ref[0], x_ref[1]], axis=1)
    ss = jnp.concatenate([ss_ref[0], ss_ref[1]], axis=1)
    nmean = ss * _rdeg(dg_ref)[:, None]
    act = _dense(x, nmean, Ws_ref, Wn_ref, sc_ref, sh_ref)
    out_ref[0] = act[:, :128]
    out_ref[1] = act[:, 128:]


def _layer3_body(x_ref, ss_ref, dg_ref, Ws_ref, Wn_ref,
                 sc_ref, sh_ref, out_ref):
    i = pl.program_id(0)
    x = jnp.concatenate([x_ref[0], x_ref[1]], axis=1)
    ss = jnp.concatenate([ss_ref[0], ss_ref[1]], axis=1)
    nmean = ss * _rdeg(dg_ref)[:, None]
    act = _dense(x, nmean, Ws_ref, Wn_ref, sc_ref, sh_ref)
    valid = (i * _BLK + lax.broadcasted_iota(jnp.int32, (_BLK, 1), 0)) < N_NODES
    act = jnp.where(valid, act, 0.0)

    @pl.when(i == 0)
    def _init():
        out_ref[...] = jnp.zeros_like(out_ref)

    out_ref[...] += jnp.sum(act, axis=0, keepdims=True)


def _stk_spec(c):
    return pl.BlockSpec((2, _BLK, c), lambda i: (0, i, 0))


def _wspecs(hin, hout):
    return [
        pl.BlockSpec((2, _BLK), lambda i: (0, i)),
        pl.BlockSpec((hin, hout), lambda i: (0, 0)),
        pl.BlockSpec((hin, hout), lambda i: (0, 0)),
        pl.BlockSpec((1, hout), lambda i: (0, 0)),
        pl.BlockSpec((1, hout), lambda i: (0, 0)),
    ]


_OUT3 = jax.ShapeDtypeStruct((2, N_PAD, 128), jnp.float32)


def _tc_layer1(x, ss, deg, Ws, Wn, scale, shift):
    return pl.pallas_call(
        _layer1_body,
        grid=(_GRID,),
        in_specs=[pl.BlockSpec((_BLK, 128), lambda i: (i, 0)), _stk_spec(128)]
        + _wspecs(128, 256),
        out_specs=_stk_spec(128),
        out_shape=_OUT3,
    )(x, ss, deg, Ws, Wn, scale, shift)


def _tc_layer2(x, ss, deg, Ws, Wn, scale, shift):
    return pl.pallas_call(
        _layer2_body,
        grid=(_GRID,),
        in_specs=[_stk_spec(128), _stk_spec(128)] + _wspecs(256, 256),
        out_specs=_stk_spec(128),
        out_shape=_OUT3,
    )(x, ss, deg, Ws, Wn, scale, shift)


def _tc_layer3(x, ss, deg, Ws, Wn, scale, shift):
    return pl.pallas_call(
        _layer3_body,
        grid=(_GRID,),
        in_specs=[_stk_spec(128), _stk_spec(128)] + _wspecs(256, 256),
        out_specs=pl.BlockSpec((1, 256), lambda i: (0, 0)),
        out_shape=jax.ShapeDtypeStruct((1, 256), jnp.float32),
    )(x, ss, deg, Ws, Wn, scale, shift)


def _head_body(cs_ref, w1_ref, b1_ref, w2_ref, b2_ref, w3_ref, b3_ref, o_ref):
    hg = cs_ref[...] * (1.0 / N_NODES)
    y = jnp.dot(hg, w1_ref[...], preferred_element_type=jnp.float32) + b1_ref[...]
    y = jnp.where(y >= 0, y, 0.01 * y)
    y = jnp.dot(y, w2_ref[...], preferred_element_type=jnp.float32) + b2_ref[...]
    y = jnp.where(y >= 0, y, 0.01 * y)
    o_ref[...] = jnp.dot(y, w3_ref[...], preferred_element_type=jnp.float32) + b3_ref[...]


def _head(colsum, fc1W, fc1b, fc2W, fc2b, fc3W, fc3b):
    nc = fc3W.shape[1]
    return pl.pallas_call(
        _head_body,
        out_shape=jax.ShapeDtypeStruct((1, nc), jnp.float32),
    )(colsum, fc1W, fc1b[None, :], fc2W, fc2b[None, :], fc3W, fc3b[None, :])


def _fold_bn(b, g, bb, m, v):
    scale = g / jnp.sqrt(v + 1e-5)
    shift = (b - m) * scale + bb
    return scale[None, :], shift[None, :]


def kernel(h, edge_index, Ws1, Wn1, b1, Ws2, Wn2, b2, Ws3, Wn3, b3,
           bn1g, bn1b, bn1m, bn1v, bn2g, bn2b, bn2m, bn2v,
           bn3g, bn3b, bn3m, bn3v, fc1W, fc1b, fc2W, fc2b, fc3W, fc3b):
    src = edge_index[0]
    dst = edge_index[1]
    z128 = jnp.zeros((RPS, 128), jnp.float32)

    sc1, sh1 = _fold_bn(b1, bn1g, bn1b, bn1m, bn1v)
    sc2, sh2 = _fold_bn(b2, bn2g, bn2b, bn2m, bn2v)
    sc3, sh3 = _fold_bn(b3, bn3g, bn3b, bn3m, bn3v)

    ss1f, deg1d = _agg_edge(h, src, dst, z128)
    deg = deg1d.reshape(2, N_PAD)
    ss1 = ss1f.reshape(2, N_PAD, 128)
    x1 = _tc_layer1(h, ss1, deg, Ws1, Wn1, sc1, sh1)
    ss2 = _agg_col(x1.reshape(2 * N_PAD, 128), src, dst,
                   z128).reshape(2, N_PAD, 128)
    x2 = _tc_layer2(x1, ss2, deg, Ws2, Wn2, sc2, sh2)
    ss3 = _agg_col(x2.reshape(2 * N_PAD, 128), src, dst,
                   z128).reshape(2, N_PAD, 128)
    colsum = _tc_layer3(x2, ss3, deg, Ws3, Wn3, sc3, sh3)
    return _head(colsum, fc1W, fc1b, fc2W, fc2b, fc3W, fc3b)


# speedup vs baseline: 1.0981x; 1.0981x over previous
import functools

import jax
import jax.numpy as jnp
from jax import lax
from jax.experimental import pallas as pl
from jax.experimental.pallas import tpu as pltpu
from jax.experimental.pallas import tpu_sc as plsc

N_NODES = 10000
N_PAD = 10240
N_EDGES = 320000
NCORES = 2
NSUB = 16
CHUNK = 128
RPS = N_PAD // NSUB

_MESH = dict(core_axis_name="c", subcore_axis_name="s",
             num_cores=NCORES, num_subcores=NSUB)

_E_TILE = N_EDGES // (NCORES * NSUB)
_ET_FULL = _E_TILE // CHUNK
_ET_TAIL = _E_TILE - _ET_FULL * CHUNK

_E_SUB = N_EDGES // NSUB
_ES_FULL = _E_SUB // CHUNK
_ES_TAIL = _E_SUB - _ES_FULL * CHUNK


_NB1 = 2
_ET_STEPS = _ET_FULL // _NB1


@functools.partial(
    pl.kernel,
    out_type=(jax.ShapeDtypeStruct((2 * N_PAD, 128), jnp.float32),
              jax.ShapeDtypeStruct((2 * N_PAD,), jnp.float32)),
    mesh=plsc.VectorSubcoreMesh(**_MESH),
    scratch_types=[
        [pltpu.VMEM((CHUNK,), jnp.int32)] * _NB1,
        [pltpu.VMEM((CHUNK,), jnp.int32)] * _NB1,
        [pltpu.VMEM((CHUNK, 128), jnp.float32)] * _NB1,
        pltpu.VMEM((_ET_TAIL,), jnp.int32),
        pltpu.VMEM((_ET_TAIL,), jnp.int32),
        pltpu.VMEM((_ET_TAIL, 128), jnp.float32),
        pltpu.VMEM((CHUNK,), jnp.float32),
        pltpu.VMEM((RPS,), jnp.float32),
        pltpu.VMEM_SHARED((N_PAD, 128), jnp.float32),
        pltpu.VMEM_SHARED((N_PAD,), jnp.float32),
        [pltpu.SemaphoreType.DMA] * _NB1,
        [pltpu.SemaphoreType.DMA] * _NB1,
        [pltpu.SemaphoreType.DMA] * _NB1,
    ],
)
def _agg_edge(tab, src_h, dst_h, z_h, out, dout,
              src_v, dst_v, rows_v, srct_v, dstt_v, rowst_v, ones_v, z1_v,
              acc, dacc, gsem, ssem, dsem):
    core = lax.axis_index("c")
    sub = lax.axis_index("s")
    wid = core * NSUB + sub
    zvec = jnp.zeros((16,), jnp.float32)
    ovec = jnp.ones((16,), jnp.float32)
    for j in range(CHUNK // 16):
        ones_v[pl.ds(j * 16, 16)] = ovec
    for j in range(RPS // 16):
        z1_v[pl.ds(j * 16, 16)] = zvec
    pltpu.sync_copy(z_h, acc.at[pl.ds(sub * RPS, RPS)])
    pltpu.sync_copy(z1_v, dacc.at[pl.ds(sub * RPS, RPS)])
    plsc.subcore_barrier()

    def load_and_gather(c, b):
        base = wid * _E_TILE + c * CHUNK
        pltpu.sync_copy(src_h.at[pl.ds(base, CHUNK)], src_v[b])
        pltpu.sync_copy(dst_h.at[pl.ds(base, CHUNK)], dst_v[b])
        pltpu.async_copy(tab.at[src_v[b]], rows_v[b], gsem[b])

    for b in range(_NB1):
        load_and_gather(b, b)

    def step(p, carry):
        for b in range(_NB1):
            c = p * _NB1 + b
            pltpu.make_async_copy(tab.at[src_v[b]], rows_v[b], gsem[b]).wait()
            pltpu.async_copy(rows_v[b], acc.at[dst_v[b]], ssem[b], add=True)
            pltpu.async_copy(ones_v, dacc.at[dst_v[b]], dsem[b], add=True)

            @pl.when(p < _ET_STEPS - 1)
            def _prep():
                pltpu.make_async_copy(
                    ones_v, dacc.at[dst_v[b]], dsem[b]).wait()
                pltpu.make_async_copy(
                    rows_v[b], acc.at[dst_v[b]], ssem[b]).wait()
                load_and_gather(c + _NB1, b)

        return carry

    lax.fori_loop(0, _ET_STEPS, step, 0)
    base = wid * _E_TILE + _ET_FULL * CHUNK
    pltpu.sync_copy(src_h.at[pl.ds(base, _ET_TAIL)], srct_v)
    pltpu.sync_copy(dst_h.at[pl.ds(base, _ET_TAIL)], dstt_v)
    pltpu.async_copy(tab.at[srct_v], rowst_v, gsem[0]).wait()
    pltpu.sync_copy(rowst_v, acc.at[dstt_v], add=True)
    pltpu.sync_copy(ones_v.at[pl.ds(0, _ET_TAIL)], dacc.at[dstt_v], add=True)
    for b in range(_NB1):
        pltpu.make_async_copy(ones_v, dacc.at[dst_v[b]], dsem[b]).wait()
        pltpu.make_async_copy(rows_v[b], acc.at[dst_v[b]], ssem[b]).wait()
    plsc.subcore_barrier()
    pltpu.sync_copy(acc.at[pl.ds(sub * RPS, RPS)],
                    out.at[pl.ds(core * N_PAD + sub * RPS, RPS)])
    pltpu.sync_copy(dacc.at[pl.ds(sub * RPS, RPS)],
                    dout.at[pl.ds(core * N_PAD + sub * RPS, RPS)])


_NB2 = 2
_ES_STEPS = _ES_FULL // _NB2


@functools.partial(
    pl.kernel,
    out_type=jax.ShapeDtypeStruct((2 * N_PAD, 128), jnp.float32),
    mesh=plsc.VectorSubcoreMesh(**_MESH),
    scratch_types=[
        [pltpu.VMEM((CHUNK,), jnp.int32)] * _NB2,
        [pltpu.VMEM((CHUNK,), jnp.int32)] * _NB2,
        [pltpu.VMEM((CHUNK, 128), jnp.float32)] * _NB2,
        pltpu.VMEM((_ES_TAIL,), jnp.int32),
        pltpu.VMEM((_ES_TAIL,), jnp.int32),
        pltpu.VMEM((_ES_TAIL, 128), jnp.float32),
        pltpu.VMEM_SHARED((N_PAD, 128), jnp.float32),
        [pltpu.SemaphoreType.DMA] * _NB2,
        [pltpu.SemaphoreType.DMA] * _NB2,
    ],
)
def _agg_col(tab, src_h, dst_h, z_h, out,
             src_v, dst_v, rows_v, srct_v, dstt_v, rowst_v, acc, gsem, ssem):
    core = lax.axis_index("c")
    sub = lax.axis_index("s")
    toff = core * N_PAD
    pltpu.sync_copy(z_h, acc.at[pl.ds(sub * RPS, RPS)])
    plsc.subcore_barrier()

    def load_and_gather(c, b):
        base = sub * _E_SUB + c * CHUNK
        pltpu.sync_copy(src_h.at[pl.ds(base, CHUNK)], src_v[b])
        pltpu.sync_copy(dst_h.at[pl.ds(base, CHUNK)], dst_v[b])
        for j in range(CHUNK // 16):
            sl = pl.ds(j * 16, 16)
            src_v[b][sl] = src_v[b][sl] + toff
        pltpu.async_copy(tab.at[src_v[b]], rows_v[b], gsem[b])

    for b in range(_NB2):
        load_and_gather(b, b)

    def step(p, carry):
        for b in range(_NB2):
            c = p * _NB2 + b
            pltpu.make_async_copy(tab.at[src_v[b]], rows_v[b], gsem[b]).wait()
            pltpu.async_copy(rows_v[b], acc.at[dst_v[b]], ssem[b], add=True)

            @pl.when(p < _ES_STEPS - 1)
            def _prep():
                pltpu.make_async_copy(
                    rows_v[b], acc.at[dst_v[b]], ssem[b]).wait()
                load_and_gather(c + _NB2, b)

        return carry

    lax.fori_loop(0, _ES_STEPS, step, 0)
    base = sub * _E_SUB + _ES_FULL * CHUNK
    pltpu.sync_copy(src_h.at[pl.ds(base, _ES_TAIL)], srct_v)
    pltpu.sync_copy(dst_h.at[pl.ds(base, _ES_TAIL)], dstt_v)
    for j in range(_ES_TAIL // 16):
        sl = pl.ds(j * 16, 16)
        srct_v[sl] = srct_v[sl] + toff
    pltpu.async_copy(tab.at[srct_v], rowst_v, gsem[0]).wait()
    pltpu.sync_copy(rowst_v, acc.at[dstt_v], add=True)
    for b in range(_NB2):
        pltpu.make_async_copy(rows_v[b], acc.at[dst_v[b]], ssem[b]).wait()
    plsc.subcore_barrier()
    pltpu.sync_copy(acc.at[pl.ds(sub * RPS, RPS)],
                    out.at[pl.ds(core * N_PAD + sub * RPS, RPS)])


_BLK = 640
_GRID = N_PAD // _BLK


def _dense(x, nmean, Ws_ref, Wn_ref, sc_ref, sh_ref):
    y = (jnp.dot(x, Ws_ref[...], preferred_element_type=jnp.float32)
         + jnp.dot(nmean, Wn_ref[...], preferred_element_type=jnp.float32))
    y = y * sc_ref[...] + sh_ref[...]
    return jnp.where(y >= 0, y, 0.01 * y)


def _rdeg(dg_ref):
    deg = dg_ref[0, :] + dg_ref[1, :]
    return 1.0 / jnp.maximum(deg, 1.0)


def _layer1_body(x_ref, ss_ref, dg_ref, Ws_ref, Wn_ref,
                 sc_ref, sh_ref, out_ref):
    nmean = (ss_ref[0] + ss_ref[1]) * _rdeg(dg_ref)[:, None]
    act = _dense(x_ref[...], nmean, Ws_ref, Wn_ref, sc_ref, sh_ref)
    out_ref[0] = act[:, :128]
    out_ref[1] = act[:, 128:]


def _layer2_body(x_ref, ss_ref, dg_ref, Ws_ref, Wn_ref,
                 sc_ref, sh_ref, out_ref):
    x = jnp.concatenate([x_ref[0], x_ref[1]], axis=1)
    ss = jnp.concatenate([ss_ref[0], ss_ref[1]], axis=1)
    nmean = ss * _rdeg(dg_ref)[:, None]
    act = _dense(x, nmean, Ws_ref, Wn_ref, sc_ref, sh_ref)
    out_ref[0] = act[:, :128]
    out_ref[1] = act[:, 128:]


def _layer3_body(x_ref, ss_ref, dg_ref, Ws_ref, Wn_ref,
                 sc_ref, sh_ref, out_ref):
    i = pl.program_id(0)
    x = jnp.concatenate([x_ref[0], x_ref[1]], axis=1)
    ss = jnp.concatenate([ss_ref[0], ss_ref[1]], axis=1)
    nmean = ss * _rdeg(dg_ref)[:, None]
    act = _dense(x, nmean, Ws_ref, Wn_ref, sc_ref, sh_ref)
    valid = (i * _BLK + lax.broadcasted_iota(jnp.int32, (_BLK, 1), 0)) < N_NODES
    act = jnp.where(valid, act, 0.0)

    @pl.when(i == 0)
    def _init():
        out_ref[...] = jnp.zeros_like(out_ref)

    out_ref[...] += jnp.sum(act, axis=0, keepdims=True)


def _stk_spec(c):
    return pl.BlockSpec((2, _BLK, c), lambda i: (0, i, 0))


def _wspecs(hin, hout):
    return [
        pl.BlockSpec((2, _BLK), lambda i: (0, i)),
        pl.BlockSpec((hin, hout), lambda i: (0, 0)),
        pl.BlockSpec((hin, hout), lambda i: (0, 0)),
        pl.BlockSpec((1, hout), lambda i: (0, 0)),
        pl.BlockSpec((1, hout), lambda i: (0, 0)),
    ]


_OUT3 = jax.ShapeDtypeStruct((2, N_PAD, 128), jnp.float32)


def _tc_layer1(x, ss, deg, Ws, Wn, scale, shift):
    return pl.pallas_call(
        _layer1_body,
        grid=(_GRID,),
        in_specs=[pl.BlockSpec((_BLK, 128), lambda i: (i, 0)), _stk_spec(128)]
        + _wspecs(128, 256),
        out_specs=_stk_spec(128),
        out_shape=_OUT3,
    )(x, ss, deg, Ws, Wn, scale, shift)


def _tc_layer2(x, ss, deg, Ws, Wn, scale, shift):
    return pl.pallas_call(
        _layer2_body,
        grid=(_GRID,),
        in_specs=[_stk_spec(128), _stk_spec(128)] + _wspecs(256, 256),
        out_specs=_stk_spec(128),
        out_shape=_OUT3,
    )(x, ss, deg, Ws, Wn, scale, shift)


def _tc_layer3(x, ss, deg, Ws, Wn, scale, shift):
    return pl.pallas_call(
        _layer3_body,
        grid=(_GRID,),
        in_specs=[_stk_spec(128), _stk_spec(128)] + _wspecs(256, 256),
        out_specs=pl.BlockSpec((1, 256), lambda i: (0, 0)),
        out_shape=jax.ShapeDtypeStruct((1, 256), jnp.float32),
    )(x, ss, deg, Ws, Wn, scale, shift)


def _head_body(cs_ref, w1_ref, b1_ref, w2_ref, b2_ref, w3_ref, b3_ref, o_ref):
    hg = cs_ref[...] * (1.0 / N_NODES)
    y = jnp.dot(hg, w1_ref[...], preferred_element_type=jnp.float32) + b1_ref[...]
    y = jnp.where(y >= 0, y, 0.01 * y)
    y = jnp.dot(y, w2_ref[...], preferred_element_type=jnp.float32) + b2_ref[...]
    y = jnp.where(y >= 0, y, 0.01 * y)
    o_ref[...] = jnp.dot(y, w3_ref[...], preferred_element_type=jnp.float32) + b3_ref[...]


def _head(colsum, fc1W, fc1b, fc2W, fc2b, fc3W, fc3b):
    nc = fc3W.shape[1]
    return pl.pallas_call(
        _head_body,
        out_shape=jax.ShapeDtypeStruct((1, nc), jnp.float32),
    )(colsum, fc1W, fc1b[None, :], fc2W, fc2b[None, :], fc3W, fc3b[None, :])


def _fold_bn(b, g, bb, m, v):
    scale = g / jnp.sqrt(v + 1e-5)
    shift = (b - m) * scale + bb
    return scale[None, :], shift[None, :]


def kernel(h, edge_index, Ws1, Wn1, b1, Ws2, Wn2, b2, Ws3, Wn3, b3,
           bn1g, bn1b, bn1m, bn1v, bn2g, bn2b, bn2m, bn2v,
           bn3g, bn3b, bn3m, bn3v, fc1W, fc1b, fc2W, fc2b, fc3W, fc3b):
    src = edge_index[0]
    dst = edge_index[1]
    z128 = jnp.zeros((RPS, 128), jnp.float32)

    sc1, sh1 = _fold_bn(b1, bn1g, bn1b, bn1m, bn1v)
    sc2, sh2 = _fold_bn(b2, bn2g, bn2b, bn2m, bn2v)
    sc3, sh3 = _fold_bn(b3, bn3g, bn3b, bn3m, bn3v)

    ss1f, deg1d = _agg_edge(h, src, dst, z128)
    deg = deg1d.reshape(2, N_PAD)
    ss1 = ss1f.reshape(2, N_PAD, 128)
    x1 = _tc_layer1(h, ss1, deg, Ws1, Wn1, sc1, sh1)
    ss2 = _agg_col(x1.reshape(2 * N_PAD, 128), src, dst,
                   z128).reshape(2, N_PAD, 128)
    x2 = _tc_layer2(x1, ss2, deg, Ws2, Wn2, sc2, sh2)
    ss3 = _agg_col(x2.reshape(2 * N_PAD, 128), src, dst,
                   z128).reshape(2, N_PAD, 128)
    colsum = _tc_layer3(x2, ss3, deg, Ws3, Wn3, sc3, sh3)
    return _head(colsum, fc1W, fc1b, fc2W, fc2b, fc3W, fc3b)


# speedup vs baseline: 1.1170x; 1.0172x over previous
import functools

import jax
import jax.numpy as jnp
from jax import lax
from jax.experimental import pallas as pl
from jax.experimental.pallas import tpu as pltpu
from jax.experimental.pallas import tpu_sc as plsc

N_NODES = 10000
N_PAD = 10240
N_EDGES = 320000
NCORES = 2
NSUB = 16
CHUNK = 128
RPS = N_PAD // NSUB

_MESH = dict(core_axis_name="c", subcore_axis_name="s",
             num_cores=NCORES, num_subcores=NSUB)

_E_TILE = N_EDGES // (NCORES * NSUB)
_ET_FULL = _E_TILE // CHUNK
_ET_TAIL = _E_TILE - _ET_FULL * CHUNK

_E_SUB = N_EDGES // NSUB
_ES_FULL = _E_SUB // CHUNK
_ES_TAIL = _E_SUB - _ES_FULL * CHUNK


_NB1 = 2
_ET_STEPS = _ET_FULL // _NB1


@functools.partial(
    pl.kernel,
    out_type=(jax.ShapeDtypeStruct((2 * N_PAD, 128), jnp.float32),
              jax.ShapeDtypeStruct((2 * N_PAD,), jnp.float32)),
    mesh=plsc.VectorSubcoreMesh(**_MESH),
    scratch_types=[
        [pltpu.VMEM((CHUNK,), jnp.int32)] * _NB1,
        [pltpu.VMEM((CHUNK,), jnp.int32)] * _NB1,
        [pltpu.VMEM((CHUNK, 128), jnp.float32)] * _NB1,
        pltpu.VMEM((_ET_TAIL,), jnp.int32),
        pltpu.VMEM((_ET_TAIL,), jnp.int32),
        pltpu.VMEM((_ET_TAIL, 128), jnp.float32),
        pltpu.VMEM((CHUNK,), jnp.float32),
        pltpu.VMEM((RPS,), jnp.float32),
        pltpu.VMEM_SHARED((N_PAD, 128), jnp.float32),
        pltpu.VMEM_SHARED((N_PAD,), jnp.float32),
        [pltpu.SemaphoreType.DMA] * _NB1,
        [pltpu.SemaphoreType.DMA] * _NB1,
        [pltpu.SemaphoreType.DMA] * _NB1,
    ],
)
def _agg_edge(tab, src_h, dst_h, z_h, out, dout,
              src_v, dst_v, rows_v, srct_v, dstt_v, rowst_v, ones_v, z1_v,
              acc, dacc, gsem, ssem, dsem):
    core = lax.axis_index("c")
    sub = lax.axis_index("s")
    wid = core * NSUB + sub
    zvec = jnp.zeros((16,), jnp.float32)
    ovec = jnp.ones((16,), jnp.float32)
    for j in range(CHUNK // 16):
        ones_v[pl.ds(j * 16, 16)] = ovec
    for j in range(RPS // 16):
        z1_v[pl.ds(j * 16, 16)] = zvec
    pltpu.sync_copy(z_h, acc.at[pl.ds(sub * RPS, RPS)])
    pltpu.sync_copy(z1_v, dacc.at[pl.ds(sub * RPS, RPS)])
    plsc.subcore_barrier()

    def load_and_gather(c, b):
        base = wid * _E_TILE + c * CHUNK
        pltpu.sync_copy(src_h.at[pl.ds(base, CHUNK)], src_v[b])
        pltpu.sync_copy(dst_h.at[pl.ds(base, CHUNK)], dst_v[b])
        pltpu.async_copy(tab.at[src_v[b]], rows_v[b], gsem[b])

    for b in range(_NB1):
        load_and_gather(b, b)

    def step(p, carry):
        for b in range(_NB1):
            c = p * _NB1 + b
            pltpu.make_async_copy(tab.at[src_v[b]], rows_v[b], gsem[b]).wait()
            pltpu.async_copy(rows_v[b], acc.at[dst_v[b]], ssem[b], add=True)
            pltpu.async_copy(ones_v, dacc.at[dst_v[b]], dsem[b], add=True)

            @pl.when(p < _ET_STEPS - 1)
            def _prep():
                pltpu.make_async_copy(
                    ones_v, dacc.at[dst_v[b]], dsem[b]).wait()
                pltpu.make_async_copy(
                    rows_v[b], acc.at[dst_v[b]], ssem[b]).wait()
                load_and_gather(c + _NB1, b)

        return carry

    lax.fori_loop(0, _ET_STEPS, step, 0)
    base = wid * _E_TILE + _ET_FULL * CHUNK
    pltpu.sync_copy(src_h.at[pl.ds(base, _ET_TAIL)], srct_v)
    pltpu.sync_copy(dst_h.at[pl.ds(base, _ET_TAIL)], dstt_v)
    pltpu.async_copy(tab.at[srct_v], rowst_v, gsem[0]).wait()
    pltpu.sync_copy(rowst_v, acc.at[dstt_v], add=True)
    pltpu.sync_copy(ones_v.at[pl.ds(0, _ET_TAIL)], dacc.at[dstt_v], add=True)
    for b in range(_NB1):
        pltpu.make_async_copy(ones_v, dacc.at[dst_v[b]], dsem[b]).wait()
        pltpu.make_async_copy(rows_v[b], acc.at[dst_v[b]], ssem[b]).wait()
    plsc.subcore_barrier()
    pltpu.sync_copy(acc.at[pl.ds(sub * RPS, RPS)],
                    out.at[pl.ds(core * N_PAD + sub * RPS, RPS)])
    pltpu.sync_copy(dacc.at[pl.ds(sub * RPS, RPS)],
                    dout.at[pl.ds(core * N_PAD + sub * RPS, RPS)])


_NB2 = 2
_ES_STEPS = _ES_FULL // _NB2


@functools.partial(
    pl.kernel,
    out_type=jax.ShapeDtypeStruct((2 * N_PAD, 128), jnp.float32),
    mesh=plsc.VectorSubcoreMesh(**_MESH),
    scratch_types=[
        [pltpu.VMEM((CHUNK,), jnp.int32)] * _NB2,
        [pltpu.VMEM((CHUNK,), jnp.int32)] * _NB2,
        [pltpu.VMEM((CHUNK, 128), jnp.float32)] * _NB2,
        pltpu.VMEM((_ES_TAIL,), jnp.int32),
        pltpu.VMEM((_ES_TAIL,), jnp.int32),
        pltpu.VMEM((_ES_TAIL, 128), jnp.float32),
        pltpu.VMEM_SHARED((N_PAD, 128), jnp.float32),
        [pltpu.SemaphoreType.DMA] * _NB2,
        [pltpu.SemaphoreType.DMA] * _NB2,
    ],
)
def _agg_col(tab, src_h, dst_h, z_h, out,
             src_v, dst_v, rows_v, srct_v, dstt_v, rowst_v, acc, gsem, ssem):
    core = lax.axis_index("c")
    sub = lax.axis_index("s")
    toff = core * N_PAD
    pltpu.sync_copy(z_h, acc.at[pl.ds(sub * RPS, RPS)])
    plsc.subcore_barrier()

    def load_and_gather(c, b):
        base = sub * _E_SUB + c * CHUNK
        pltpu.sync_copy(src_h.at[pl.ds(base, CHUNK)], src_v[b])
        pltpu.sync_copy(dst_h.at[pl.ds(base, CHUNK)], dst_v[b])
        for j in range(CHUNK // 16):
            sl = pl.ds(j * 16, 16)
            src_v[b][sl] = src_v[b][sl] + toff
        pltpu.async_copy(tab.at[src_v[b]], rows_v[b], gsem[b])

    for b in range(_NB2):
        load_and_gather(b, b)

    def step(p, carry):
        for b in range(_NB2):
            c = p * _NB2 + b
            pltpu.make_async_copy(tab.at[src_v[b]], rows_v[b], gsem[b]).wait()
            pltpu.async_copy(rows_v[b], acc.at[dst_v[b]], ssem[b], add=True)

            @pl.when(p < _ES_STEPS - 1)
            def _prep():
                pltpu.make_async_copy(
                    rows_v[b], acc.at[dst_v[b]], ssem[b]).wait()
                load_and_gather(c + _NB2, b)

        return carry

    lax.fori_loop(0, _ES_STEPS, step, 0)
    base = sub * _E_SUB + _ES_FULL * CHUNK
    pltpu.sync_copy(src_h.at[pl.ds(base, _ES_TAIL)], srct_v)
    pltpu.sync_copy(dst_h.at[pl.ds(base, _ES_TAIL)], dstt_v)
    for j in range(_ES_TAIL // 16):
        sl = pl.ds(j * 16, 16)
        srct_v[sl] = srct_v[sl] + toff
    pltpu.async_copy(tab.at[srct_v], rowst_v, gsem[0]).wait()
    pltpu.sync_copy(rowst_v, acc.at[dstt_v], add=True)
    for b in range(_NB2):
        pltpu.make_async_copy(rows_v[b], acc.at[dst_v[b]], ssem[b]).wait()
    plsc.subcore_barrier()
    pltpu.sync_copy(acc.at[pl.ds(sub * RPS, RPS)],
                    out.at[pl.ds(core * N_PAD + sub * RPS, RPS)])


_BLK = 1280
_GRID = N_PAD // _BLK


def _dense(x, nmean, Ws_ref, Wn_ref, sc_ref, sh_ref):
    y = (jnp.dot(x, Ws_ref[...], preferred_element_type=jnp.float32)
         + jnp.dot(nmean, Wn_ref[...], preferred_element_type=jnp.float32))
    y = y * sc_ref[...] + sh_ref[...]
    return jnp.where(y >= 0, y, 0.01 * y)


def _rdeg(dg_ref):
    deg = dg_ref[0, :] + dg_ref[1, :]
    return 1.0 / jnp.maximum(deg, 1.0)


def _layer1_body(x_ref, ss_ref, dg_ref, Ws_ref, Wn_ref,
                 sc_ref, sh_ref, out_ref):
    nmean = (ss_ref[0] + ss_ref[1]) * _rdeg(dg_ref)[:, None]
    act = _dense(x_ref[...], nmean, Ws_ref, Wn_ref, sc_ref, sh_ref)
    out_ref[0] = act[:, :128]
    out_ref[1] = act[:, 128:]


def _layer2_body(x_ref, ss_ref, dg_ref, Ws_ref, Wn_ref,
                 sc_ref, sh_ref, out_ref):
    x = jnp.concatenate([x_ref[0], x_ref[1]], axis=1)
    ss = jnp.concatenate([ss_ref[0], ss_ref[1]], axis=1)
    nmean = ss * _rdeg(dg_ref)[:, None]
    act = _dense(x, nmean, Ws_ref, Wn_ref, sc_ref, sh_ref)
    out_ref[0] = act[:, :128]
    out_ref[1] = act[:, 128:]


def _layer3_body(x_ref, ss_ref, dg_ref, Ws_ref, Wn_ref,
                 sc_ref, sh_ref, out_ref):
    i = pl.program_id(0)
    x = jnp.concatenate([x_ref[0], x_ref[1]], axis=1)
    ss = jnp.concatenate([ss_ref[0], ss_ref[1]], axis=1)
    nmean = ss * _rdeg(dg_ref)[:, None]
    act = _dense(x, nmean, Ws_ref, Wn_ref, sc_ref, sh_ref)
    valid = (i * _BLK + lax.broadcasted_iota(jnp.int32, (_BLK, 1), 0)) < N_NODES
    act = jnp.where(valid, act, 0.0)

    @pl.when(i == 0)
    def _init():
        out_ref[...] = jnp.zeros_like(out_ref)

    out_ref[...] += jnp.sum(act, axis=0, keepdims=True)


def _stk_spec(c):
    return pl.BlockSpec((2, _BLK, c), lambda i: (0, i, 0))


def _wspecs(hin, hout):
    return [
        pl.BlockSpec((2, _BLK), lambda i: (0, i)),
        pl.BlockSpec((hin, hout), lambda i: (0, 0)),
        pl.BlockSpec((hin, hout), lambda i: (0, 0)),
        pl.BlockSpec((1, hout), lambda i: (0, 0)),
        pl.BlockSpec((1, hout), lambda i: (0, 0)),
    ]


_OUT3 = jax.ShapeDtypeStruct((2, N_PAD, 128), jnp.float32)


def _tc_layer1(x, ss, deg, Ws, Wn, scale, shift):
    return pl.pallas_call(
        _layer1_body,
        grid=(_GRID,),
        in_specs=[pl.BlockSpec((_BLK, 128), lambda i: (i, 0)), _stk_spec(128)]
        + _wspecs(128, 256),
        out_specs=_stk_spec(128),
        out_shape=_OUT3,
    )(x, ss, deg, Ws, Wn, scale, shift)


def _tc_layer2(x, ss, deg, Ws, Wn, scale, shift):
    return pl.pallas_call(
        _layer2_body,
        grid=(_GRID,),
        in_specs=[_stk_spec(128), _stk_spec(128)] + _wspecs(256, 256),
        out_specs=_stk_spec(128),
        out_shape=_OUT3,
    )(x, ss, deg, Ws, Wn, scale, shift)


def _tc_layer3(x, ss, deg, Ws, Wn, scale, shift):
    return pl.pallas_call(
        _layer3_body,
        grid=(_GRID,),
        in_specs=[_stk_spec(128), _stk_spec(128)] + _wspecs(256, 256),
        out_specs=pl.BlockSpec((1, 256), lambda i: (0, 0)),
        out_shape=jax.ShapeDtypeStruct((1, 256), jnp.float32),
    )(x, ss, deg, Ws, Wn, scale, shift)


def _head_body(cs_ref, w1_ref, b1_ref, w2_ref, b2_ref, w3_ref, b3_ref, o_ref):
    hg = cs_ref[...] * (1.0 / N_NODES)
    y = jnp.dot(hg, w1_ref[...], preferred_element_type=jnp.float32) + b1_ref[...]
    y = jnp.where(y >= 0, y, 0.01 * y)
    y = jnp.dot(y, w2_ref[...], preferred_element_type=jnp.float32) + b2_ref[...]
    y = jnp.where(y >= 0, y, 0.01 * y)
    o_ref[...] = jnp.dot(y, w3_ref[...], preferred_element_type=jnp.float32) + b3_ref[...]


def _head(colsum, fc1W, fc1b, fc2W, fc2b, fc3W, fc3b):
    nc = fc3W.shape[1]
    return pl.pallas_call(
        _head_body,
        out_shape=jax.ShapeDtypeStruct((1, nc), jnp.float32),
    )(colsum, fc1W, fc1b[None, :], fc2W, fc2b[None, :], fc3W, fc3b[None, :])


def _fold_bn(b, g, bb, m, v):
    scale = g / jnp.sqrt(v + 1e-5)
    shift = (b - m) * scale + bb
    return scale[None, :], shift[None, :]


def kernel(h, edge_index, Ws1, Wn1, b1, Ws2, Wn2, b2, Ws3, Wn3, b3,
           bn1g, bn1b, bn1m, bn1v, bn2g, bn2b, bn2m, bn2v,
           bn3g, bn3b, bn3m, bn3v, fc1W, fc1b, fc2W, fc2b, fc3W, fc3b):
    src = edge_index[0]
    dst = edge_index[1]
    z128 = jnp.zeros((RPS, 128), jnp.float32)

    sc1, sh1 = _fold_bn(b1, bn1g, bn1b, bn1m, bn1v)
    sc2, sh2 = _fold_bn(b2, bn2g, bn2b, bn2m, bn2v)
    sc3, sh3 = _fold_bn(b3, bn3g, bn3b, bn3m, bn3v)

    ss1f, deg1d = _agg_edge(h, src, dst, z128)
    deg = deg1d.reshape(2, N_PAD)
    ss1 = ss1f.reshape(2, N_PAD, 128)
    x1 = _tc_layer1(h, ss1, deg, Ws1, Wn1, sc1, sh1)
    ss2 = _agg_col(x1.reshape(2 * N_PAD, 128), src, dst,
                   z128).reshape(2, N_PAD, 128)
    x2 = _tc_layer2(x1, ss2, deg, Ws2, Wn2, sc2, sh2)
    ss3 = _agg_col(x2.reshape(2 * N_PAD, 128), src, dst,
                   z128).reshape(2, N_PAD, 128)
    colsum = _tc_layer3(x2, ss3, deg, Ws3, Wn3, sc3, sh3)
    return _head(colsum, fc1W, fc1b, fc2W, fc2b, fc3W, fc3b)


# speedup vs baseline: 1.1262x; 1.0083x over previous
import functools

import jax
import jax.numpy as jnp
from jax import lax
from jax.experimental import pallas as pl
from jax.experimental.pallas import tpu as pltpu
from jax.experimental.pallas import tpu_sc as plsc

N_NODES = 10000
N_PAD = 10240
N_EDGES = 320000
NCORES = 2
NSUB = 16
CHUNK = 128
RPS = N_PAD // NSUB

_MESH = dict(core_axis_name="c", subcore_axis_name="s",
             num_cores=NCORES, num_subcores=NSUB)

_E_TILE = N_EDGES // (NCORES * NSUB)
_ET_FULL = _E_TILE // CHUNK
_ET_TAIL = _E_TILE - _ET_FULL * CHUNK

_E_SUB = N_EDGES // NSUB
_ES_FULL = _E_SUB // CHUNK
_ES_TAIL = _E_SUB - _ES_FULL * CHUNK


_NB1 = 2
_ET_STEPS = _ET_FULL // _NB1


@functools.partial(
    pl.kernel,
    out_type=(jax.ShapeDtypeStruct((2 * N_PAD, 128), jnp.float32),
              jax.ShapeDtypeStruct((2 * N_PAD,), jnp.float32)),
    mesh=plsc.VectorSubcoreMesh(**_MESH),
    scratch_types=[
        [pltpu.VMEM((CHUNK,), jnp.int32)] * _NB1,
        [pltpu.VMEM((CHUNK,), jnp.int32)] * _NB1,
        [pltpu.VMEM((CHUNK, 128), jnp.float32)] * _NB1,
        pltpu.VMEM((_ET_TAIL,), jnp.int32),
        pltpu.VMEM((_ET_TAIL,), jnp.int32),
        pltpu.VMEM((_ET_TAIL, 128), jnp.float32),
        pltpu.VMEM((CHUNK,), jnp.float32),
        pltpu.VMEM((RPS,), jnp.float32),
        pltpu.VMEM_SHARED((N_PAD, 128), jnp.float32),
        pltpu.VMEM_SHARED((N_PAD,), jnp.float32),
        [pltpu.SemaphoreType.DMA] * _NB1,
        [pltpu.SemaphoreType.DMA] * _NB1,
        [pltpu.SemaphoreType.DMA] * _NB1,
    ],
)
def _agg_edge(tab, src_h, dst_h, z_h, out, dout,
              src_v, dst_v, rows_v, srct_v, dstt_v, rowst_v, ones_v, z1_v,
              acc, dacc, gsem, ssem, dsem):
    core = lax.axis_index("c")
    sub = lax.axis_index("s")
    wid = core * NSUB + sub
    zvec = jnp.zeros((16,), jnp.float32)
    ovec = jnp.ones((16,), jnp.float32)
    for j in range(CHUNK // 16):
        ones_v[pl.ds(j * 16, 16)] = ovec
    for j in range(RPS // 16):
        z1_v[pl.ds(j * 16, 16)] = zvec
    pltpu.sync_copy(z_h, acc.at[pl.ds(sub * RPS, RPS)])
    pltpu.sync_copy(z1_v, dacc.at[pl.ds(sub * RPS, RPS)])
    plsc.subcore_barrier()

    def load_and_gather(c, b):
        base = wid * _E_TILE + c * CHUNK
        pltpu.sync_copy(src_h.at[pl.ds(base, CHUNK)], src_v[b])
        pltpu.sync_copy(dst_h.at[pl.ds(base, CHUNK)], dst_v[b])
        pltpu.async_copy(tab.at[src_v[b]], rows_v[b], gsem[b])

    for b in range(_NB1):
        load_and_gather(b, b)

    def step(p, carry):
        for b in range(_NB1):
            c = p * _NB1 + b
            pltpu.make_async_copy(tab.at[src_v[b]], rows_v[b], gsem[b]).wait()
            pltpu.async_copy(rows_v[b], acc.at[dst_v[b]], ssem[b], add=True)
            pltpu.async_copy(ones_v, dacc.at[dst_v[b]], dsem[b], add=True)

            @pl.when(p < _ET_STEPS - 1)
            def _prep():
                pltpu.make_async_copy(
                    ones_v, dacc.at[dst_v[b]], dsem[b]).wait()
                pltpu.make_async_copy(
                    rows_v[b], acc.at[dst_v[b]], ssem[b]).wait()
                load_and_gather(c + _NB1, b)

        return carry

    lax.fori_loop(0, _ET_STEPS, step, 0)
    base = wid * _E_TILE + _ET_FULL * CHUNK
    pltpu.sync_copy(src_h.at[pl.ds(base, _ET_TAIL)], srct_v)
    pltpu.sync_copy(dst_h.at[pl.ds(base, _ET_TAIL)], dstt_v)
    pltpu.async_copy(tab.at[srct_v], rowst_v, gsem[0]).wait()
    pltpu.sync_copy(rowst_v, acc.at[dstt_v], add=True)
    pltpu.sync_copy(ones_v.at[pl.ds(0, _ET_TAIL)], dacc.at[dstt_v], add=True)
    for b in range(_NB1):
        pltpu.make_async_copy(ones_v, dacc.at[dst_v[b]], dsem[b]).wait()
        pltpu.make_async_copy(rows_v[b], acc.at[dst_v[b]], ssem[b]).wait()
    plsc.subcore_barrier()
    pltpu.sync_copy(acc.at[pl.ds(sub * RPS, RPS)],
                    out.at[pl.ds(core * N_PAD + sub * RPS, RPS)])
    pltpu.sync_copy(dacc.at[pl.ds(sub * RPS, RPS)],
                    dout.at[pl.ds(core * N_PAD + sub * RPS, RPS)])


_NB2 = 2
_ES_STEPS = _ES_FULL // _NB2


@functools.partial(
    pl.kernel,
    out_type=jax.ShapeDtypeStruct((2 * N_PAD, 128), jnp.float32),
    mesh=plsc.VectorSubcoreMesh(**_MESH),
    scratch_types=[
        [pltpu.VMEM((CHUNK,), jnp.int32)] * _NB2,
        [pltpu.VMEM((CHUNK,), jnp.int32)] * _NB2,
        [pltpu.VMEM((CHUNK, 128), jnp.float32)] * _NB2,
        pltpu.VMEM((_ES_TAIL,), jnp.int32),
        pltpu.VMEM((_ES_TAIL,), jnp.int32),
        pltpu.VMEM((_ES_TAIL, 128), jnp.float32),
        pltpu.VMEM_SHARED((N_PAD, 128), jnp.float32),
        [pltpu.SemaphoreType.DMA] * _NB2,
        [pltpu.SemaphoreType.DMA] * _NB2,
    ],
)
def _agg_col(tab, src_h, dst_h, z_h, out,
             src_v, dst_v, rows_v, srct_v, dstt_v, rowst_v, acc, gsem, ssem):
    core = lax.axis_index("c")
    sub = lax.axis_index("s")
    toff = core * N_PAD
    pltpu.sync_copy(z_h, acc.at[pl.ds(sub * RPS, RPS)])
    plsc.subcore_barrier()

    def load_and_gather(c, b):
        base = sub * _E_SUB + c * CHUNK
        pltpu.sync_copy(src_h.at[pl.ds(base, CHUNK)], src_v[b])
        pltpu.sync_copy(dst_h.at[pl.ds(base, CHUNK)], dst_v[b])
        for j in range(CHUNK // 16):
            sl = pl.ds(j * 16, 16)
            src_v[b][sl] = src_v[b][sl] + toff
        pltpu.async_copy(tab.at[src_v[b]], rows_v[b], gsem[b])

    for b in range(_NB2):
        load_and_gather(b, b)

    def step(p, carry):
        for b in range(_NB2):
            c = p * _NB2 + b
            pltpu.make_async_copy(tab.at[src_v[b]], rows_v[b], gsem[b]).wait()
            pltpu.async_copy(rows_v[b], acc.at[dst_v[b]], ssem[b], add=True)

            @pl.when(p < _ES_STEPS - 1)
            def _prep():
                pltpu.make_async_copy(
                    rows_v[b], acc.at[dst_v[b]], ssem[b]).wait()
                load_and_gather(c + _NB2, b)

        return carry

    lax.fori_loop(0, _ES_STEPS, step, 0)
    base = sub * _E_SUB + _ES_FULL * CHUNK
    pltpu.sync_copy(src_h.at[pl.ds(base, _ES_TAIL)], srct_v)
    pltpu.sync_copy(dst_h.at[pl.ds(base, _ES_TAIL)], dstt_v)
    for j in range(_ES_TAIL // 16):
        sl = pl.ds(j * 16, 16)
        srct_v[sl] = srct_v[sl] + toff
    pltpu.async_copy(tab.at[srct_v], rowst_v, gsem[0]).wait()
    pltpu.sync_copy(rowst_v, acc.at[dstt_v], add=True)
    for b in range(_NB2):
        pltpu.make_async_copy(rows_v[b], acc.at[dst_v[b]], ssem[b]).wait()
    plsc.subcore_barrier()
    pltpu.sync_copy(acc.at[pl.ds(sub * RPS, RPS)],
                    out.at[pl.ds(core * N_PAD + sub * RPS, RPS)])


_BLK = 2560
_GRID = N_PAD // _BLK


def _dense(x, nmean, Ws_ref, Wn_ref, sc_ref, sh_ref):
    y = (jnp.dot(x, Ws_ref[...], preferred_element_type=jnp.float32)
         + jnp.dot(nmean, Wn_ref[...], preferred_element_type=jnp.float32))
    y = y * sc_ref[...] + sh_ref[...]
    return jnp.where(y >= 0, y, 0.01 * y)


def _rdeg(dg_ref):
    deg = dg_ref[0, :] + dg_ref[1, :]
    return 1.0 / jnp.maximum(deg, 1.0)


def _layer1_body(x_ref, ss_ref, dg_ref, Ws_ref, Wn_ref,
                 sc_ref, sh_ref, out_ref):
    nmean = (ss_ref[0] + ss_ref[1]) * _rdeg(dg_ref)[:, None]
    act = _dense(x_ref[...], nmean, Ws_ref, Wn_ref, sc_ref, sh_ref)
    out_ref[0] = act[:, :128]
    out_ref[1] = act[:, 128:]


def _layer2_body(x_ref, ss_ref, dg_ref, Ws_ref, Wn_ref,
                 sc_ref, sh_ref, out_ref):
    x = jnp.concatenate([x_ref[0], x_ref[1]], axis=1)
    ss = jnp.concatenate([ss_ref[0], ss_ref[1]], axis=1)
    nmean = ss * _rdeg(dg_ref)[:, None]
    act = _dense(x, nmean, Ws_ref, Wn_ref, sc_ref, sh_ref)
    out_ref[0] = act[:, :128]
    out_ref[1] = act[:, 128:]


def _layer3_body(x_ref, ss_ref, dg_ref, Ws_ref, Wn_ref,
                 sc_ref, sh_ref, out_ref):
    i = pl.program_id(0)
    x = jnp.concatenate([x_ref[0], x_ref[1]], axis=1)
    ss = jnp.concatenate([ss_ref[0], ss_ref[1]], axis=1)
    nmean = ss * _rdeg(dg_ref)[:, None]
    act = _dense(x, nmean, Ws_ref, Wn_ref, sc_ref, sh_ref)
    valid = (i * _BLK + lax.broadcasted_iota(jnp.int32, (_BLK, 1), 0)) < N_NODES
    act = jnp.where(valid, act, 0.0)

    @pl.when(i == 0)
    def _init():
        out_ref[...] = jnp.zeros_like(out_ref)

    out_ref[...] += jnp.sum(act, axis=0, keepdims=True)


def _stk_spec(c):
    return pl.BlockSpec((2, _BLK, c), lambda i: (0, i, 0))


def _wspecs(hin, hout):
    return [
        pl.BlockSpec((2, _BLK), lambda i: (0, i)),
        pl.BlockSpec((hin, hout), lambda i: (0, 0)),
        pl.BlockSpec((hin, hout), lambda i: (0, 0)),
        pl.BlockSpec((1, hout), lambda i: (0, 0)),
        pl.BlockSpec((1, hout), lambda i: (0, 0)),
    ]


_OUT3 = jax.ShapeDtypeStruct((2, N_PAD, 128), jnp.float32)


def _tc_layer1(x, ss, deg, Ws, Wn, scale, shift):
    return pl.pallas_call(
        _layer1_body,
        grid=(_GRID,),
        in_specs=[pl.BlockSpec((_BLK, 128), lambda i: (i, 0)), _stk_spec(128)]
        + _wspecs(128, 256),
        out_specs=_stk_spec(128),
        out_shape=_OUT3,
    )(x, ss, deg, Ws, Wn, scale, shift)


def _tc_layer2(x, ss, deg, Ws, Wn, scale, shift):
    return pl.pallas_call(
        _layer2_body,
        grid=(_GRID,),
        in_specs=[_stk_spec(128), _stk_spec(128)] + _wspecs(256, 256),
        out_specs=_stk_spec(128),
        out_shape=_OUT3,
    )(x, ss, deg, Ws, Wn, scale, shift)


def _tc_layer3(x, ss, deg, Ws, Wn, scale, shift):
    return pl.pallas_call(
        _layer3_body,
        grid=(_GRID,),
        in_specs=[_stk_spec(128), _stk_spec(128)] + _wspecs(256, 256),
        out_specs=pl.BlockSpec((1, 256), lambda i: (0, 0)),
        out_shape=jax.ShapeDtypeStruct((1, 256), jnp.float32),
    )(x, ss, deg, Ws, Wn, scale, shift)


def _head_body(cs_ref, w1_ref, b1_ref, w2_ref, b2_ref, w3_ref, b3_ref, o_ref):
    hg = cs_ref[...] * (1.0 / N_NODES)
    y = jnp.dot(hg, w1_ref[...], preferred_element_type=jnp.float32) + b1_ref[...]
    y = jnp.where(y >= 0, y, 0.01 * y)
    y = jnp.dot(y, w2_ref[...], preferred_element_type=jnp.float32) + b2_ref[...]
    y = jnp.where(y >= 0, y, 0.01 * y)
    o_ref[...] = jnp.dot(y, w3_ref[...], preferred_element_type=jnp.float32) + b3_ref[...]


def _head(colsum, fc1W, fc1b, fc2W, fc2b, fc3W, fc3b):
    nc = fc3W.shape[1]
    return pl.pallas_call(
        _head_body,
        out_shape=jax.ShapeDtypeStruct((1, nc), jnp.float32),
    )(colsum, fc1W, fc1b[None, :], fc2W, fc2b[None, :], fc3W, fc3b[None, :])


def _fold_bn(b, g, bb, m, v):
    scale = g / jnp.sqrt(v + 1e-5)
    shift = (b - m) * scale + bb
    return scale[None, :], shift[None, :]


def kernel(h, edge_index, Ws1, Wn1, b1, Ws2, Wn2, b2, Ws3, Wn3, b3,
           bn1g, bn1b, bn1m, bn1v, bn2g, bn2b, bn2m, bn2v,
           bn3g, bn3b, bn3m, bn3v, fc1W, fc1b, fc2W, fc2b, fc3W, fc3b):
    src = edge_index[0]
    dst = edge_index[1]
    z128 = jnp.zeros((RPS, 128), jnp.float32)

    sc1, sh1 = _fold_bn(b1, bn1g, bn1b, bn1m, bn1v)
    sc2, sh2 = _fold_bn(b2, bn2g, bn2b, bn2m, bn2v)
    sc3, sh3 = _fold_bn(b3, bn3g, bn3b, bn3m, bn3v)

    ss1f, deg1d = _agg_edge(h, src, dst, z128)
    deg = deg1d.reshape(2, N_PAD)
    ss1 = ss1f.reshape(2, N_PAD, 128)
    x1 = _tc_layer1(h, ss1, deg, Ws1, Wn1, sc1, sh1)
    ss2 = _agg_col(x1.reshape(2 * N_PAD, 128), src, dst,
                   z128).reshape(2, N_PAD, 128)
    x2 = _tc_layer2(x1, ss2, deg, Ws2, Wn2, sc2, sh2)
    ss3 = _agg_col(x2.reshape(2 * N_PAD, 128), src, dst,
                   z128).reshape(2, N_PAD, 128)
    colsum = _tc_layer3(x2, ss3, deg, Ws3, Wn3, sc3, sh3)
    return _head(colsum, fc1W, fc1b, fc2W, fc2b, fc3W, fc3b)
